# baseline (speedup 1.0000x reference)
import functools

import jax
import jax.numpy as jnp
from jax import lax
from jax.experimental import pallas as pl
from jax.experimental.pallas import tpu as pltpu
from jax.experimental.pallas import tpu_sc as plsc

_L = 16
_CH = 128
_NB = 4


@functools.cache
def _build(NI, NJ, V, D):
    info = plsc.get_sparse_core_info()
    NC, NS = info.num_cores, info.num_subcores
    NW = NC * NS
    NP = _CH // 2
    NT = 2 * NJ
    assert NI == NW * _CH and NJ == 200 and D == 64
    mesh = plsc.VectorSubcoreMesh(core_axis_name="c", subcore_axis_name="s")

    @functools.partial(
        pl.kernel,
        out_type=jax.ShapeDtypeStruct((NI, NJ, D), jnp.float32),
        mesh=mesh,
        scratch_types=[
            pltpu.VMEM((NJ // 8, 8, _CH), jnp.int32),
            pltpu.VMEM((_NB, NT), jnp.int32),
            pltpu.VMEM((_NB, 2, NJ, D), jnp.float32),
            pltpu.SemaphoreType.DMA,
            pltpu.SemaphoreType.DMA,
        ],
        compiler_params=pltpu.CompilerParams(
            use_tc_tiling_on_sc=False, needs_layout_passes=False),
    )
    def gather_kernel(xv_hbm, tab_hbm, out_hbm, idx_v, rid_v, gbuf,
                      gsem, ssem):
        wid = lax.axis_index("s") * NC + lax.axis_index("c")
        i0 = wid * _CH
        for tr in range(NJ // 8):
            pltpu.sync_copy(xv_hbm.at[pl.ds(tr * 8 * NW + wid * 8, 8)],
                            idx_v.at[tr])
        lanes = lax.iota(jnp.int32, _L)

        def fill_and_gather(k, b):
            l0 = 2 * k
            for g in range(NT // _L):
                pv = lanes + (_L * g)
                if _L * (g + 1) <= NJ:
                    jv, lv = pv, lanes * 0 + l0
                elif _L * g >= NJ:
                    jv, lv = pv - NJ, lanes * 0 + (l0 + 1)
                else:
                    oi = jnp.where(pv >= NJ, 1, 0).astype(jnp.int32)
                    jv, lv = pv - NJ * oi, oi + l0
                tr = lax.shift_right_logical(jv, 3)
                v = plsc.load_gather(idx_v, [tr, jv & 7, lv])
                rid_v[b, pl.ds(_L * g, _L)] = v
            for h in range(2):
                for off, n in ((0, 128), (128, NJ - 128)):
                    pltpu.async_copy(
                        tab_hbm.at[rid_v.at[b, pl.ds(NJ * h + off, n)]],
                        gbuf.at[b, h, pl.ds(off, n)], gsem)

        for b in range(2):
            fill_and_gather(jnp.int32(b), b)

        def step(k, carry):
            for b in range(_NB):
                kk = _NB * k + b
                for _h in range(2):
                    for off, n in ((0, 128), (128, NJ - 128)):
                        pltpu.make_async_copy(
                            tab_hbm.at[pl.ds(0, n)],
                            gbuf.at[0, 0, pl.ds(off, n)], gsem).wait()
                for h in range(2):
                    pltpu.async_copy(
                        gbuf.at[b, h], out_hbm.at[i0 + 2 * kk + h], ssem)

                @pl.when(kk >= 2)
                def _():
                    for _h in range(2):
                        pltpu.make_async_copy(
                            gbuf.at[0, 0], out_hbm.at[0], ssem).wait()

                @pl.when(kk + 2 < NP)
                def _():
                    fill_and_gather(kk + 2, (kk + 2) % _NB)

            return carry

        assert NP % _NB == 0
        lax.fori_loop(0, NP // _NB, step, 0)
        for _ in range(4):
            pltpu.make_async_copy(
                gbuf.at[0, 0], out_hbm.at[0], ssem).wait()

    return gather_kernel


def kernel(x, table):
    S0, S1 = x.shape
    V, D = table.shape
    xv = (x.astype(jnp.int32)
          .reshape(S0 // _CH, _CH, S1 // 8, 8)
          .transpose(2, 0, 3, 1)
          .reshape(S1 // 8 * (S0 // _CH) * 8, _CH))
    return _build(S0, S1, V, D)(xv, table)


# speedup vs baseline: 1.0108x; 1.0108x over previous
import functools

import jax
import jax.numpy as jnp
from jax import lax
from jax.experimental import pallas as pl
from jax.experimental.pallas import tpu as pltpu
from jax.experimental.pallas import tpu_sc as plsc

_CH = 128
_NBUF = 8
_S = _NBUF // 2


@functools.cache
def _build(B, D):
    info = plsc.get_sparse_core_info()
    NC, NS = info.num_cores, info.num_subcores
    NW = NC * NS
    b_per_w = B // NW
    n_chunks = b_per_w // _CH
    n_groups = n_chunks // _NBUF
    assert B % NW == 0 and b_per_w % _CH == 0 and n_chunks % _NBUF == 0
    assert n_groups >= 2
    mesh = plsc.VectorSubcoreMesh(core_axis_name="c", subcore_axis_name="s")

    @functools.partial(
        pl.kernel,
        out_type=jax.ShapeDtypeStruct((B, D), jnp.float32),
        mesh=mesh,
        scratch_types=[
            pltpu.VMEM((n_chunks, _CH), jnp.int32),
            pltpu.VMEM((_NBUF, _CH, D), jnp.float32),
            pltpu.SemaphoreType.DMA,
            pltpu.SemaphoreType.DMA,
        ],
        compiler_params=pltpu.CompilerParams(use_tc_tiling_on_sc=False),
    )
    def gather_kernel(idx_hbm, table_hbm, out_hbm, idx_v, rows_v, gsem, ssem):
        wid = lax.axis_index("s") * NC + lax.axis_index("c")
        base_row = wid * n_chunks
        pltpu.sync_copy(idx_hbm.at[pl.ds(base_row, n_chunks)], idx_v)

        def start_gather(j, b):
            pltpu.async_copy(table_hbm.at[idx_v.at[j]], rows_v.at[b], gsem)

        def wait_gather(b):
            pltpu.make_async_copy(
                table_hbm.at[pl.ds(0, _CH)], rows_v.at[b], gsem).wait()

        def start_store(j, b):
            pltpu.async_copy(
                rows_v.at[b], out_hbm.at[pl.ds((base_row + j) * _CH, _CH)], ssem)

        def wait_store(b):
            pltpu.make_async_copy(
                rows_v.at[b], out_hbm.at[pl.ds(0, _CH)], ssem).wait()

        for b in range(_NBUF):
            start_gather(b, b)

        for b in range(_NBUF):
            wait_gather(b)
            start_store(b, b)
            if b >= _S:
                wait_store(b - _S)
                start_gather(b + _S, b - _S)

        def group(g, carry):
            j0 = g * _NBUF
            for b in range(_NBUF):
                j = j0 + b
                wait_gather(b)
                start_store(j, b)
                wait_store((b + _S) % _NBUF)
                start_gather(j + _S, (b + _S) % _NBUF)
            return carry

        lax.fori_loop(1, n_groups - 1, group, 0)

        j0 = (n_groups - 1) * _NBUF
        for b in range(_NBUF):
            j = j0 + b
            wait_gather(b)
            start_store(j, b)
            wait_store((b + _S) % _NBUF)
            if b < _S:
                start_gather(j + _S, (b + _S) % _NBUF)
        for b in range(_S):
            wait_store(b)

    return gather_kernel


def kernel(x, table):
    S0, S1 = x.shape
    B = S0 * S1
    D = table.shape[1]
    idx = x.reshape(B // _CH, _CH).astype(jnp.int32)
    out = _build(B, D)(idx, table)
    return out.reshape(S0, S1, D)
